# baseline (device time: 196962 ns/iter reference)
import jax
import jax.numpy as jnp
from jax import lax
from jax.experimental import pallas as pl
from jax.experimental.pallas import tpu as pltpu

N_DEV = 8
N_SUB = 4


def _mod(v):
    return lax.rem(v + 2 * N_DEV, N_DEV)


def kernel(x):
    m, n = x.shape
    ch = m // N_DEV
    hn = n // 2
    hr = ch // N_SUB
    xb = x.astype(jnp.bfloat16)

    def body(x_ref, out_ref, comm_r, comm_l,
             rs_send_r, rs_recv_r, rs_send_l, rs_recv_l,
             ag_send_r, ag_recv_r, ag_send_l, ag_recv_l):
        me = lax.axis_index("i")
        right = _mod(me + 1)
        left = _mod(me - 1)

        def rs_r(s, j):
            cs = _mod(me - s)
            src = x_ref if s == 0 else out_ref
            return pltpu.make_async_remote_copy(
                src_ref=src.at[pl.ds(cs * ch + j * hr, hr), 0:hn],
                dst_ref=comm_r.at[s, pl.ds(j * hr, hr), :],
                send_sem=rs_send_r.at[s, j],
                recv_sem=rs_recv_r.at[s, j],
                device_id=(right,),
                device_id_type=pl.DeviceIdType.MESH,
            )

        def rs_l(s, j):
            cs = _mod(me + s)
            src = x_ref if s == 0 else out_ref
            return pltpu.make_async_remote_copy(
                src_ref=src.at[pl.ds(cs * ch + j * hr, hr), hn:n],
                dst_ref=comm_l.at[s, pl.ds(j * hr, hr), :],
                send_sem=rs_send_l.at[s, j],
                recv_sem=rs_recv_l.at[s, j],
                device_id=(left,),
                device_id_type=pl.DeviceIdType.MESH,
            )

        def ag_r(t, j):
            cs = _mod(me + 1 - t)
            sl = (pl.ds(cs * ch + j * hr, hr), slice(0, hn))
            return pltpu.make_async_remote_copy(
                src_ref=out_ref.at[sl],
                dst_ref=out_ref.at[sl],
                send_sem=ag_send_r.at[t, j],
                recv_sem=ag_recv_r.at[t, j],
                device_id=(right,),
                device_id_type=pl.DeviceIdType.MESH,
            )

        def ag_l(t, j):
            cs = _mod(me - 1 + t)
            sl = (pl.ds(cs * ch + j * hr, hr), slice(hn, n))
            return pltpu.make_async_remote_copy(
                src_ref=out_ref.at[sl],
                dst_ref=out_ref.at[sl],
                send_sem=ag_send_l.at[t, j],
                recv_sem=ag_recv_l.at[t, j],
                device_id=(left,),
                device_id_type=pl.DeviceIdType.MESH,
            )

        barrier_sem = pltpu.get_barrier_semaphore()
        for nbr in (left, right):
            pl.semaphore_signal(
                barrier_sem, inc=1,
                device_id=(nbr,), device_id_type=pl.DeviceIdType.MESH,
            )
        pl.semaphore_wait(barrier_sem, 2)

        for j in range(N_SUB):
            rs_r(0, j).start()
            rs_l(0, j).start()
        for s in range(N_DEV - 1):
            cr_r = _mod(me - s - 1)
            cr_l = _mod(me + s + 1)
            for j in range(N_SUB):
                row_r = pl.ds(cr_r * ch + j * hr, hr)
                rs_r(s, j).wait_recv()
                out_ref[row_r, 0:hn] = (
                    x_ref[row_r, 0:hn]
                    + comm_r[s, pl.ds(j * hr, hr), :]
                )
                if s < N_DEV - 2:
                    rs_r(s + 1, j).start()
                row_l = pl.ds(cr_l * ch + j * hr, hr)
                rs_l(s, j).wait_recv()
                out_ref[row_l, hn:n] = (
                    x_ref[row_l, hn:n]
                    + comm_l[s, pl.ds(j * hr, hr), :]
                )
                if s < N_DEV - 2:
                    rs_l(s + 1, j).start()

        for j in range(N_SUB):
            ag_r(0, j).start()
            ag_l(0, j).start()
        for t in range(N_DEV - 1):
            for j in range(N_SUB):
                ag_r(t, j).wait_recv()
                if t < N_DEV - 2:
                    ag_r(t + 1, j).start()
                ag_l(t, j).wait_recv()
                if t < N_DEV - 2:
                    ag_l(t + 1, j).start()

        for s in range(N_DEV - 1):
            for j in range(N_SUB):
                rs_r(s, j).wait_send()
                rs_l(s, j).wait_send()
                ag_r(s, j).wait_send()
                ag_l(s, j).wait_send()

    sems = pltpu.SemaphoreType.DMA((N_DEV - 1, N_SUB))
    return pl.pallas_call(
        body,
        out_shape=jax.ShapeDtypeStruct((m, n), jnp.bfloat16),
        in_specs=[pl.BlockSpec(memory_space=pltpu.VMEM)],
        out_specs=pl.BlockSpec(memory_space=pltpu.VMEM),
        scratch_shapes=[
            pltpu.VMEM((N_DEV - 1, ch, hn), jnp.bfloat16),
            pltpu.VMEM((N_DEV - 1, ch, hn), jnp.bfloat16),
            sems, sems, sems, sems,
            sems, sems, sems, sems,
        ],
        compiler_params=pltpu.CompilerParams(collective_id=0),
    )(xb)


# device time: 173782 ns/iter; 1.1334x vs baseline; 1.1334x over previous
import jax
import jax.numpy as jnp
from jax import lax
from jax.experimental import pallas as pl
from jax.experimental.pallas import tpu as pltpu

N_DEV = 8
N_SUB = 2

_CONV_OFFS = (0, -1, 1, -2, 2, -3, 3, 4)


def _mod(v):
    return lax.rem(v + 2 * N_DEV, N_DEV)


def kernel(x):
    m, n = x.shape
    ch = m // N_DEV
    hn = n // 2
    hr = ch // N_SUB

    def body(x_ref, out_ref, stage, comm_r, comm_l, load_sems,
             rs_send_r, rs_recv_r, rs_send_l, rs_recv_l,
             ag_send_r, ag_recv_r, ag_send_l, ag_recv_l):
        me = lax.axis_index("i")
        right = _mod(me + 1)
        left = _mod(me - 1)

        def load(i):
            c = _mod(me + _CONV_OFFS[i])
            return pltpu.make_async_copy(
                x_ref.at[pl.ds(c * ch, ch), :],
                stage.at[i % 2],
                load_sems.at[i],
            )

        def conv(i):
            c = _mod(me + _CONV_OFFS[i])
            out_ref[pl.ds(c * ch, ch), :] = stage[i % 2].astype(
                jnp.bfloat16
            )

        def rs_r(s, j):
            cs = _mod(me - s)
            return pltpu.make_async_remote_copy(
                src_ref=out_ref.at[pl.ds(cs * ch + j * hr, hr), 0:hn],
                dst_ref=comm_r.at[s, pl.ds(j * hr, hr), :],
                send_sem=rs_send_r.at[s, j],
                recv_sem=rs_recv_r.at[s, j],
                device_id=(right,),
                device_id_type=pl.DeviceIdType.MESH,
            )

        def rs_l(s, j):
            cs = _mod(me + s)
            return pltpu.make_async_remote_copy(
                src_ref=out_ref.at[pl.ds(cs * ch + j * hr, hr), hn:n],
                dst_ref=comm_l.at[s, pl.ds(j * hr, hr), :],
                send_sem=rs_send_l.at[s, j],
                recv_sem=rs_recv_l.at[s, j],
                device_id=(left,),
                device_id_type=pl.DeviceIdType.MESH,
            )

        def ag_r(t, j):
            cs = _mod(me + 1 - t)
            sl = (pl.ds(cs * ch + j * hr, hr), slice(0, hn))
            return pltpu.make_async_remote_copy(
                src_ref=out_ref.at[sl],
                dst_ref=out_ref.at[sl],
                send_sem=ag_send_r.at[t, j],
                recv_sem=ag_recv_r.at[t, j],
                device_id=(right,),
                device_id_type=pl.DeviceIdType.MESH,
            )

        def ag_l(t, j):
            cs = _mod(me - 1 + t)
            sl = (pl.ds(cs * ch + j * hr, hr), slice(hn, n))
            return pltpu.make_async_remote_copy(
                src_ref=out_ref.at[sl],
                dst_ref=out_ref.at[sl],
                send_sem=ag_send_l.at[t, j],
                recv_sem=ag_recv_l.at[t, j],
                device_id=(left,),
                device_id_type=pl.DeviceIdType.MESH,
            )

        load(0).start()
        load(1).start()
        barrier_sem = pltpu.get_barrier_semaphore()
        for nbr in (left, right):
            pl.semaphore_signal(
                barrier_sem, inc=1,
                device_id=(nbr,), device_id_type=pl.DeviceIdType.MESH,
            )
        pl.semaphore_wait(barrier_sem, 2)

        load(0).wait()
        conv(0)
        for j in range(N_SUB):
            rs_r(0, j).start()
            rs_l(0, j).start()
        load(2).start()
        load(1).wait()
        conv(1)
        load(3).start()
        load(2).wait()
        conv(2)
        load(4).start()

        for s in range(N_DEV - 1):
            if 1 <= s <= 3:
                for i in (2 * s + 1, 2 * s + 2):
                    if i < len(_CONV_OFFS):
                        load(i).wait()
                        conv(i)
                        if i + 2 < len(_CONV_OFFS):
                            load(i + 2).start()
            cr_r = _mod(me - s - 1)
            cr_l = _mod(me + s + 1)
            for j in range(N_SUB):
                row_r = pl.ds(cr_r * ch + j * hr, hr)
                rs_r(s, j).wait_recv()
                out_ref[row_r, 0:hn] = (
                    out_ref[row_r, 0:hn]
                    + comm_r[s, pl.ds(j * hr, hr), :]
                )
                if s < N_DEV - 2:
                    rs_r(s + 1, j).start()
                row_l = pl.ds(cr_l * ch + j * hr, hr)
                rs_l(s, j).wait_recv()
                out_ref[row_l, hn:n] = (
                    out_ref[row_l, hn:n]
                    + comm_l[s, pl.ds(j * hr, hr), :]
                )
                if s < N_DEV - 2:
                    rs_l(s + 1, j).start()

        for j in range(N_SUB):
            ag_r(0, j).start()
            ag_l(0, j).start()
        for t in range(N_DEV - 1):
            for j in range(N_SUB):
                ag_r(t, j).wait_recv()
                if t < N_DEV - 2:
                    ag_r(t + 1, j).start()
                ag_l(t, j).wait_recv()
                if t < N_DEV - 2:
                    ag_l(t + 1, j).start()

        for s in range(N_DEV - 1):
            for j in range(N_SUB):
                rs_r(s, j).wait_send()
                rs_l(s, j).wait_send()
                ag_r(s, j).wait_send()
                ag_l(s, j).wait_send()

    sems = pltpu.SemaphoreType.DMA((N_DEV - 1, N_SUB))
    return pl.pallas_call(
        body,
        out_shape=jax.ShapeDtypeStruct((m, n), jnp.bfloat16),
        in_specs=[pl.BlockSpec(memory_space=pltpu.MemorySpace.HBM)],
        out_specs=pl.BlockSpec(memory_space=pltpu.VMEM),
        scratch_shapes=[
            pltpu.VMEM((2, ch, n), jnp.float32),
            pltpu.VMEM((N_DEV - 1, ch, hn), jnp.bfloat16),
            pltpu.VMEM((N_DEV - 1, ch, hn), jnp.bfloat16),
            pltpu.SemaphoreType.DMA((N_DEV,)),
            sems, sems, sems, sems,
            sems, sems, sems, sems,
        ],
        compiler_params=pltpu.CompilerParams(collective_id=0),
    )(x)


# device time: 171474 ns/iter; 1.1486x vs baseline; 1.0135x over previous
import jax
import jax.numpy as jnp
from jax import lax
from jax.experimental import pallas as pl
from jax.experimental.pallas import tpu as pltpu

N_DEV = 8
N_SUB = 2

_CONV_OFFS = (0, -1, 1, -2, 2, -3, 3, 4)


def _mod(v):
    return lax.rem(v + 2 * N_DEV, N_DEV)


def kernel(x):
    m, n = x.shape
    ch = m // N_DEV
    hn = n // 2
    hr = ch // N_SUB

    def body(x_ref, out_ref, stage, comm_r, comm_l, load_sems,
             rs_send_r, rs_recv_r, rs_send_l, rs_recv_l,
             ag_send_r, ag_recv_r, ag_send_l, ag_recv_l):
        me = lax.axis_index("i")
        right = _mod(me + 1)
        left = _mod(me - 1)

        def load(i):
            c = _mod(me + _CONV_OFFS[i])
            return pltpu.make_async_copy(
                x_ref.at[pl.ds(c * ch, ch), :],
                stage.at[i % 2],
                load_sems.at[i],
            )

        def conv(i):
            c = _mod(me + _CONV_OFFS[i])
            out_ref[pl.ds(c * ch, ch), :] = stage[i % 2].astype(
                jnp.bfloat16
            )

        def rs_r(s, j):
            cs = _mod(me - s)
            return pltpu.make_async_remote_copy(
                src_ref=out_ref.at[pl.ds(cs * ch + j * hr, hr), 0:hn],
                dst_ref=comm_r.at[s, pl.ds(j * hr, hr), :],
                send_sem=rs_send_r.at[s, j],
                recv_sem=rs_recv_r.at[s, j],
                device_id=(right,),
                device_id_type=pl.DeviceIdType.MESH,
            )

        def rs_l(s, j):
            cs = _mod(me + s)
            return pltpu.make_async_remote_copy(
                src_ref=out_ref.at[pl.ds(cs * ch + j * hr, hr), hn:n],
                dst_ref=comm_l.at[s, pl.ds(j * hr, hr), :],
                send_sem=rs_send_l.at[s, j],
                recv_sem=rs_recv_l.at[s, j],
                device_id=(left,),
                device_id_type=pl.DeviceIdType.MESH,
            )

        def ag_r(t, j):
            cs = _mod(me + 1 - t)
            sl = (pl.ds(cs * ch + j * hr, hr), slice(0, hn))
            return pltpu.make_async_remote_copy(
                src_ref=out_ref.at[sl],
                dst_ref=out_ref.at[sl],
                send_sem=ag_send_r.at[t, j],
                recv_sem=ag_recv_r.at[t, j],
                device_id=(right,),
                device_id_type=pl.DeviceIdType.MESH,
            )

        def ag_l(t, j):
            cs = _mod(me - 1 + t)
            sl = (pl.ds(cs * ch + j * hr, hr), slice(hn, n))
            return pltpu.make_async_remote_copy(
                src_ref=out_ref.at[sl],
                dst_ref=out_ref.at[sl],
                send_sem=ag_send_l.at[t, j],
                recv_sem=ag_recv_l.at[t, j],
                device_id=(left,),
                device_id_type=pl.DeviceIdType.MESH,
            )

        load(0).start()
        load(1).start()
        barrier_sem = pltpu.get_barrier_semaphore()
        for nbr in (left, right):
            pl.semaphore_signal(
                barrier_sem, inc=1,
                device_id=(nbr,), device_id_type=pl.DeviceIdType.MESH,
            )
        pl.semaphore_wait(barrier_sem, 2)

        load(0).wait()
        c0 = _mod(me)
        for j in range(N_SUB):
            out_ref[pl.ds(c0 * ch + j * hr, hr), :] = stage[
                0, pl.ds(j * hr, hr), :
            ].astype(jnp.bfloat16)
            rs_r(0, j).start()
            rs_l(0, j).start()
        load(2).start()
        load(1).wait()
        conv(1)
        load(3).start()
        load(2).wait()
        conv(2)
        load(4).start()

        def ldconv(i):
            if i < len(_CONV_OFFS):
                load(i).wait()
                conv(i)
                if i + 2 < len(_CONV_OFFS):
                    load(i + 2).start()

        for s in range(N_DEV - 1):
            cr_r = _mod(me - s - 1)
            cr_l = _mod(me + s + 1)
            if 1 <= s <= 3:
                ldconv(2 * s + 1)
            for j in range(N_SUB):
                row_r = pl.ds(cr_r * ch + j * hr, hr)
                rs_r(s, j).wait_recv()
                out_ref[row_r, 0:hn] = (
                    out_ref[row_r, 0:hn]
                    + comm_r[s, pl.ds(j * hr, hr), :]
                )
                if s < N_DEV - 2:
                    rs_r(s + 1, j).start()
                elif s == N_DEV - 2:
                    ag_r(0, j).start()
                if j == 0 and 1 <= s <= 3:
                    ldconv(2 * s + 2)
                row_l = pl.ds(cr_l * ch + j * hr, hr)
                rs_l(s, j).wait_recv()
                out_ref[row_l, hn:n] = (
                    out_ref[row_l, hn:n]
                    + comm_l[s, pl.ds(j * hr, hr), :]
                )
                if s < N_DEV - 2:
                    rs_l(s + 1, j).start()
                elif s == N_DEV - 2:
                    ag_l(0, j).start()

        for t in range(N_DEV - 1):
            for j in range(N_SUB):
                ag_r(t, j).wait_recv()
                if t < N_DEV - 2:
                    ag_r(t + 1, j).start()
                ag_l(t, j).wait_recv()
                if t < N_DEV - 2:
                    ag_l(t + 1, j).start()

        for s in range(N_DEV - 1):
            for j in range(N_SUB):
                rs_r(s, j).wait_send()
                rs_l(s, j).wait_send()
                ag_r(s, j).wait_send()
                ag_l(s, j).wait_send()

    sems = pltpu.SemaphoreType.DMA((N_DEV - 1, N_SUB))
    return pl.pallas_call(
        body,
        out_shape=jax.ShapeDtypeStruct((m, n), jnp.bfloat16),
        in_specs=[pl.BlockSpec(memory_space=pltpu.MemorySpace.HBM)],
        out_specs=pl.BlockSpec(memory_space=pltpu.VMEM),
        scratch_shapes=[
            pltpu.VMEM((2, ch, n), jnp.float32),
            pltpu.VMEM((N_DEV - 1, ch, hn), jnp.bfloat16),
            pltpu.VMEM((N_DEV - 1, ch, hn), jnp.bfloat16),
            pltpu.SemaphoreType.DMA((N_DEV,)),
            sems, sems, sems, sems,
            sems, sems, sems, sems,
        ],
        compiler_params=pltpu.CompilerParams(collective_id=0),
    )(x)
